# baseline (speedup 1.0000x reference)
import jax
import jax.numpy as jnp
from jax import lax
from jax.experimental import pallas as pl
from jax.experimental.pallas import tpu as pltpu
from jax.experimental.pallas import tpu_sc as plsc

B, S, L, V, D = 1024, 200, 8, 100000, 128
NC, NS = 2, 16
NW = NC * NS
BPW = B // NW
LANES = 16
H0, H1 = 128, S - 128
NBUF = 6
NFUS = 3


def _body(table_hbm, ids_hbm, starts_hbm, fusion_hbm, out_hbm,
          ids_v, starts_v, fus_v, rows_v, sem_g, sem_w):
    wid = lax.axis_index("s") * NC + lax.axis_index("c")
    base = wid * BPW
    pltpu.sync_copy(ids_hbm.at[pl.ds(base, BPW)], ids_v)
    pltpu.sync_copy(starts_hbm.at[pl.ds(base, BPW)], starts_v)

    col_iota = lax.iota(jnp.int32, LANES)
    zeros16 = jnp.full((LANES,), 0, jnp.int32)

    def gather_half(j, h, slot, with_fus):
        size, off = (H0, 0) if h == 0 else (H1, H0)
        pltpu.async_copy(table_hbm.at[ids_v.at[j, pl.ds(off, size)]],
                         rows_v.at[slot, pl.ds(0, size)], sem_g.at[slot])
        if with_fus:
            pltpu.async_copy(fusion_hbm.at[base + j],
                             fus_v.at[lax.rem(j, NFUS)], sem_g.at[slot])

    def wait_gather(slot, h):
        size = H0 if h == 0 else H1
        pltpu.make_async_copy(table_hbm.at[pl.ds(0, size)],
                              rows_v.at[slot, pl.ds(0, size)],
                              sem_g.at[slot]).wait()
        if h == 0:
            pltpu.make_async_copy(fusion_hbm.at[0], fus_v.at[0],
                                  sem_g.at[slot]).wait()

    def write_half(j, h, slot):
        size, off = (H0, 0) if h == 0 else (H1, H0)
        pltpu.async_copy(rows_v.at[slot, pl.ds(0, size)],
                         out_hbm.at[base + j, pl.ds(off, size)],
                         sem_w.at[slot])

    def wait_write(slot, h):
        size = H0 if h == 0 else H1
        pltpu.make_async_copy(rows_v.at[0, pl.ds(0, size)],
                              out_hbm.at[0, pl.ds(0, size)],
                              sem_w.at[slot]).wait()

    def fusion_add(slot, fs, start_vec, h):
        kvec = zeros16 + slot
        for l in range(L):
            if h == 0:
                idx = start_vec + l
                mask = idx < H0
            else:
                idx = start_vec + (l - H0)
                mask = idx >= 0
            for c in range(D // LANES):
                x = fus_v[fs, l, pl.ds(c * LANES, LANES)]
                plsc.addupdate_scatter(
                    rows_v, [kvec, idx, col_iota + c * LANES], x, mask=mask)

    gather_half(jnp.int32(0), 0, jnp.int32(0), True)
    gather_half(jnp.int32(0), 1, jnp.int32(1), False)
    gather_half(jnp.int32(1), 0, jnp.int32(2), True)

    def step(j, carry):
        s0 = lax.rem(2 * j, NBUF)

        @pl.when(j <= BPW - 2)
        def _():
            s3 = lax.rem(2 * j + 3, NBUF)

            @pl.when(j >= 2)
            def _():
                wait_write(s3, 1)

            gather_half(j + 1, 1, s3, False)

        wait_gather(s0, 0)
        start_vec = plsc.load_gather(starts_v, [zeros16 + j])
        fs = lax.rem(j, NFUS)
        fusion_add(s0, fs, start_vec, 0)
        write_half(j, 0, s0)

        s1 = lax.rem(2 * j + 1, NBUF)

        @pl.when(j <= BPW - 3)
        def _():
            s4 = lax.rem(2 * j + 4, NBUF)

            @pl.when(j >= 1)
            def _():
                wait_write(s4, 0)

            gather_half(j + 2, 0, s4, True)

        wait_gather(s1, 1)
        fusion_add(s1, fs, start_vec, 1)
        write_half(j, 1, s1)
        return carry

    lax.fori_loop(0, BPW, step, 0)
    for t in range(NBUF):
        u = 2 * BPW - NBUF + t
        wait_write(jnp.int32(u % NBUF), u & 1)


_mesh = plsc.VectorSubcoreMesh(core_axis_name="c", subcore_axis_name="s")

_sc_call = pl.kernel(
    _body,
    out_type=jax.ShapeDtypeStruct((B, S, D), jnp.float32),
    mesh=_mesh,
    compiler_params=pltpu.CompilerParams(needs_layout_passes=False),
    scratch_types=[
        pltpu.VMEM((BPW, S), jnp.int32),
        pltpu.VMEM((BPW,), jnp.int32),
        pltpu.VMEM((NFUS, L, D), jnp.float32),
        pltpu.VMEM((NBUF, H0, D), jnp.float32),
        pltpu.SemaphoreType.DMA((NBUF,)),
        pltpu.SemaphoreType.DMA((NBUF,)),
    ],
)


def kernel(embed_table, input_ids, starts, fusion):
    return _sc_call(embed_table,
                    input_ids.astype(jnp.int32),
                    starts.astype(jnp.int32),
                    fusion)


# speedup vs baseline: 1.0145x; 1.0145x over previous
import jax
import jax.numpy as jnp
from jax import lax
from jax.experimental import pallas as pl
from jax.experimental.pallas import tpu as pltpu
from jax.experimental.pallas import tpu_sc as plsc

B, S, L, V, D = 1024, 200, 8, 100000, 128
NC, NS = 2, 16
NW = NC * NS
BPW = B // NW
LANES = 16
C0, C1 = 128, S - 128
NBUF = 4
PF = 2


QR = 56


def _body(table_hbm, ids_hbm, starts_hbm, fusion_hbm, out_hbm,
          ids_v, starts_v, fus_v, rows_v, stage_s, sem_g, sem_w, sem_a, sem_b):
    sid = lax.axis_index("s")
    wid = sid * NC + lax.axis_index("c")
    base = wid * BPW
    pltpu.sync_copy(ids_hbm.at[pl.ds(base, BPW)], ids_v)
    pltpu.sync_copy(starts_hbm.at[pl.ds(base, BPW)], starts_v)

    col_iota = lax.iota(jnp.int32, LANES)
    zeros16 = jnp.full((LANES,), 0, jnp.int32)

    def start_gather(j, k):
        pltpu.async_copy(table_hbm.at[ids_v.at[j, pl.ds(0, C0)]],
                         rows_v.at[k, pl.ds(0, C0)], sem_g.at[k])
        pltpu.async_copy(table_hbm.at[ids_v.at[j, pl.ds(C0, C1)]],
                         rows_v.at[k, pl.ds(C0, C1)], sem_g.at[k])
        pltpu.async_copy(fusion_hbm.at[base + j], fus_v.at[k], sem_g.at[k])

    def wait_gather(k):
        pltpu.make_async_copy(table_hbm.at[pl.ds(0, C0)],
                              rows_v.at[k, pl.ds(0, C0)], sem_g.at[k]).wait()
        pltpu.make_async_copy(table_hbm.at[pl.ds(0, C1)],
                              rows_v.at[k, pl.ds(C0, C1)], sem_g.at[k]).wait()
        pltpu.make_async_copy(fusion_hbm.at[0], fus_v.at[k], sem_g.at[k]).wait()

    def wait_write(k):
        pltpu.make_async_copy(rows_v.at[k, pl.ds(QR, S - QR)],
                              out_hbm.at[0, pl.ds(QR, S - QR)],
                              sem_w.at[k]).wait()

    def wait_a(ss):
        pltpu.make_async_copy(rows_v.at[0, pl.ds(0, QR)],
                              stage_s.at[0, 0], sem_a.at[ss]).wait()

    def wait_b(ss):
        pltpu.make_async_copy(stage_s.at[0, 0],
                              out_hbm.at[0, pl.ds(0, QR)], sem_b.at[ss]).wait()

    for p in range(PF):
        start_gather(p, p)

    def step(j, carry):
        k = lax.rem(j, NBUF)

        @pl.when(j + PF < BPW)
        def _():
            k2 = lax.rem(j + PF, NBUF)

            @pl.when(j + PF >= NBUF)
            def _():
                wait_write(k2)

            start_gather(j + PF, k2)

        wait_gather(k)

        kvec = zeros16 + k
        jvec = zeros16 + j
        start_vec = plsc.load_gather(starts_v, [jvec])
        for l in range(L):
            row_idx = start_vec + l
            for c in range(D // LANES):
                x = fus_v[k, l, pl.ds(c * LANES, LANES)]
                plsc.addupdate_scatter(
                    rows_v, [kvec, row_idx, col_iota + c * LANES], x)

        pltpu.async_copy(rows_v.at[k, pl.ds(QR, S - QR)],
                         out_hbm.at[base + j, pl.ds(QR, S - QR)], sem_w.at[k])
        ss = lax.rem(j, 2)

        @pl.when(j >= 2)
        def _():
            wait_b(ss)

        pltpu.async_copy(rows_v.at[k, pl.ds(0, QR)],
                         stage_s.at[sid, ss], sem_a.at[ss])

        @pl.when(j >= 1)
        def _():
            s2 = lax.rem(j + 1, 2)
            wait_a(s2)
            pltpu.async_copy(stage_s.at[sid, s2],
                             out_hbm.at[base + j - 1, pl.ds(0, QR)],
                             sem_b.at[s2])
        return carry

    lax.fori_loop(0, BPW, step, 0)
    last = jnp.int32(BPW - 1)
    wait_a(lax.rem(last, 2))
    pltpu.async_copy(stage_s.at[sid, lax.rem(last, 2)],
                     out_hbm.at[base + last, pl.ds(0, QR)],
                     sem_b.at[lax.rem(last, 2)])
    wait_b(lax.rem(last - 1, 2))
    wait_b(lax.rem(last, 2))
    for t in range(NBUF):
        wait_write(lax.rem(jnp.int32(BPW - NBUF + t), NBUF))


_mesh = plsc.VectorSubcoreMesh(core_axis_name="c", subcore_axis_name="s")

_sc_call = pl.kernel(
    _body,
    out_type=jax.ShapeDtypeStruct((B, S, D), jnp.float32),
    mesh=_mesh,
    compiler_params=pltpu.CompilerParams(needs_layout_passes=False),
    scratch_types=[
        pltpu.VMEM((BPW, S), jnp.int32),
        pltpu.VMEM((BPW,), jnp.int32),
        pltpu.VMEM((NBUF, L, D), jnp.float32),
        pltpu.VMEM((NBUF, S, D), jnp.float32),
        pltpu.VMEM_SHARED((NS, 2, QR, D), jnp.float32),
        pltpu.SemaphoreType.DMA((NBUF,)),
        pltpu.SemaphoreType.DMA((NBUF,)),
        pltpu.SemaphoreType.DMA((2,)),
        pltpu.SemaphoreType.DMA((2,)),
    ],
)


def kernel(embed_table, input_ids, starts, fusion):
    return _sc_call(embed_table,
                    input_ids.astype(jnp.int32),
                    starts.astype(jnp.int32),
                    fusion)
